# baseline (speedup 1.0000x reference)
import jax
import jax.numpy as jnp
from jax import lax
from jax.experimental import pallas as pl
from jax.experimental.pallas import tpu as pltpu
from jax.experimental.pallas import tpu_sc as plsc

_SHAPE = (4096, 11008)
_COLS = _SHAPE[1]
_NUMEL = _SHAPE[0] * _SHAPE[1]
_NC, _NS = 2, 16
_NW = _NC * _NS
_RROWS = _SHAPE[0] // _NW
_BROWS = 2
_BLK = _BROWS * _COLS
_NBLK = _RROWS // _BROWS
_NSETS = 4
_K = 1_000_000
_PC = 1536
_PCB = _PC + 16
_MROW = _NBLK * 16


def _vec_update(blk_v, bounce_i, bounce_f, idx16, val16, active, blk_base):
  iota = lax.iota(jnp.int32, 16)
  idx_m = jnp.where(active, idx16, -1)
  val_m = jnp.where(active, val16, 0.0)
  bounce_i[...] = idx_m
  up = plsc.load_gather(bounce_i, [jnp.minimum(iota + 1, 15)])
  dn = plsc.load_gather(bounce_i, [jnp.maximum(iota - 1, 0)])
  mask_last = active & ((iota == 15) | (up != idx_m))
  mask_first = active & ((iota == 0) | (dn != idx_m))
  t = plsc.cumsum(val_m)
  bounce_f[...] = t
  t_dn = plsc.load_gather(bounce_f, [jnp.maximum(iota - 1, 0)])
  t_dn = jnp.where(iota == 0, 0.0, t_dn)
  loc = idx_m - blk_base
  r = loc // _COLS
  c = loc - r * _COLS
  plsc.addupdate_scatter(blk_v, [r, c], t, mask=mask_last)
  plsc.addupdate_scatter(blk_v, [r, c], -t_dn, mask=mask_first)


class _BufSet:
  def __init__(self, blk, pi, pv, sem_in, sem_out, sem_pi, sem_pv):
    self.blk, self.pi, self.pv = blk, pi, pv
    self.sem_in, self.sem_out = sem_in, sem_out
    self.sem_pi, self.sem_pv = sem_pi, sem_pv


def _sc_body(tens_hbm, idx_hbm, val_hbm, meta_hbm, out_hbm, *refs):
  blks = refs[0:4]
  pis = refs[4:8]
  pvs = refs[8:12]
  ri, rv, meta_v, bounce_i, bounce_f = refs[12:17]
  sems = refs[17:]
  sets = [
      _BufSet(blks[j], pis[j], pvs[j],
              sems[4 * j], sems[4 * j + 1], sems[4 * j + 2], sems[4 * j + 3])
      for j in range(_NSETS)
  ]

  cid = lax.axis_index("c")
  sid = lax.axis_index("s")
  wid = sid * _NC + cid
  row_base = wid * _RROWS
  pltpu.sync_copy(meta_hbm.at[wid], meta_v)

  def get_se(b):
    mvec = meta_v[pl.ds(b * 16, 16)]
    return mvec[0], mvec[1]

  def pair_a(p):
    return jnp.minimum((p // 8) * 8, _K - _PCB)

  def in_copy(b, s):
    return pltpu.make_async_copy(
        tens_hbm.at[pl.ds(row_base + b * _BROWS, _BROWS), :], s.blk, s.sem_in)

  def out_copy(b, s):
    return pltpu.make_async_copy(
        s.blk, out_hbm.at[pl.ds(row_base + b * _BROWS, _BROWS), :], s.sem_out)

  def pair_copies(a, s):
    return (pltpu.make_async_copy(idx_hbm.at[pl.ds(a, _PCB)], s.pi, s.sem_pi),
            pltpu.make_async_copy(val_hbm.at[pl.ds(a, _PCB)], s.pv, s.sem_pv))

  def issue_front(b, s):
    in_copy(b, s).start()
    sb, _ = get_se(b)
    ci, cv = pair_copies(pair_a(sb), s)
    ci.start()
    cv.start()

  def consume(blk, idxb, valb, a, cstart, cend, blk_base):
    nvec = (cend - a + 15) // 16

    def vec_body(v, carry):
      o = v * 16
      idx16 = idxb[pl.ds(o, 16)]
      val16 = valb[pl.ds(o, 16)]
      pos = a + o + lax.iota(jnp.int32, 16)
      active = (pos >= cstart) & (pos < cend)
      _vec_update(blk, bounce_i, bounce_f, idx16, val16, active, blk_base)
      return carry

    lax.fori_loop(0, nvec, vec_body, 0)

  def half(b, mine, ahead):
    sb, eb = get_se(b)

    @pl.when(b >= 2)
    def _():
      out_copy(b - 2, ahead).wait()

    @pl.when(b + 2 < _NBLK)
    def _():
      issue_front(b + 2, ahead)

    ci, cv = pair_copies(0, mine)
    ci.wait()
    cv.wait()
    in_copy(b, mine).wait()

    blk_base = (row_base + b * _BROWS) * _COLS
    a0 = pair_a(sb)
    consume(mine.blk, mine.pi, mine.pv, a0, sb, jnp.minimum(eb, sb + _PC),
            blk_base)
    nchunks = (eb - sb + _PC - 1) // _PC

    def chunk_body(c, carry):
      cstart = sb + c * _PC
      cend = jnp.minimum(eb, cstart + _PC)
      ac = pair_a(cstart)
      pltpu.sync_copy(idx_hbm.at[pl.ds(ac, _PCB)], ri)
      pltpu.sync_copy(val_hbm.at[pl.ds(ac, _PCB)], rv)
      consume(mine.blk, ri, rv, ac, cstart, cend, blk_base)
      return carry

    lax.fori_loop(1, jnp.maximum(nchunks, 1), chunk_body, 0)
    out_copy(b, mine).start()

  issue_front(0, sets[0])
  issue_front(1, sets[1])

  def group_step(g, carry):
    b0 = 4 * g
    for j in range(_NSETS):
      half(b0 + j, sets[j], sets[(j + 2) % _NSETS])
    return carry

  lax.fori_loop(0, _NBLK // _NSETS, group_step, 0)
  out_copy(_NBLK - 2, sets[(_NBLK - 2) % _NSETS]).wait()
  out_copy(_NBLK - 1, sets[(_NBLK - 1) % _NSETS]).wait()


def kernel(tensor, values, indices):
  boundaries = (jnp.arange(_NW * _NBLK + 1, dtype=jnp.int32) * _BLK)
  bs = jnp.searchsorted(indices, boundaries, side="left").astype(jnp.int32)
  inter = jnp.stack([bs[:-1], bs[1:]], axis=1).reshape(_NW, _NBLK, 2)
  meta = (jnp.zeros((_NW, _NBLK, 16), dtype=jnp.int32)
          .at[:, :, :2].set(inter).reshape(_NW, _MROW))

  mesh = plsc.VectorSubcoreMesh(
      core_axis_name="c", subcore_axis_name="s",
      num_cores=_NC, num_subcores=_NS)
  scratch = (
      [pltpu.VMEM((_BROWS, _COLS), jnp.float32) for _ in range(_NSETS)]
      + [pltpu.VMEM((_PCB,), jnp.int32) for _ in range(_NSETS)]
      + [pltpu.VMEM((_PCB,), jnp.float32) for _ in range(_NSETS)]
      + [
          pltpu.VMEM((_PCB,), jnp.int32),
          pltpu.VMEM((_PCB,), jnp.float32),
          pltpu.VMEM((_MROW,), jnp.int32),
          pltpu.VMEM((16,), jnp.int32),
          pltpu.VMEM((16,), jnp.float32),
      ]
      + [pltpu.SemaphoreType.DMA for _ in range(4 * _NSETS)]
  )
  run = pl.kernel(
      _sc_body,
      out_type=jax.ShapeDtypeStruct(_SHAPE, jnp.float32),
      mesh=mesh,
      compiler_params=pltpu.CompilerParams(needs_layout_passes=False),
      scratch_types=scratch,
  )
  return run(tensor, indices, values, meta)


# speedup vs baseline: 1.5054x; 1.5054x over previous
import jax
import jax.numpy as jnp
from jax import lax
from jax.experimental import pallas as pl
from jax.experimental.pallas import tpu as pltpu
from jax.experimental.pallas import tpu_sc as plsc

_SHAPE = (4096, 11008)
_COLS = _SHAPE[1]
_NUMEL = _SHAPE[0] * _SHAPE[1]
_NC, _NS = 2, 16
_NW = _NC * _NS
_RROWS = _SHAPE[0] // _NW
_BROWS = 4
_BLK = _BROWS * _COLS
_NBLK = _RROWS // _BROWS
_NGRP = _NBLK // 2
_K = 1_000_000
_PC = 4096
_PCB = _PC + 16
_MROW = _NBLK * 16


def _vec_update(blk_v, bounce_i, bounce_f, idx16, val16, active, blk_base):
  iota = lax.iota(jnp.int32, 16)
  idx_m = jnp.where(active, idx16, -1)
  val_m = jnp.where(active, val16, 0.0)
  bounce_i[...] = idx_m
  up = plsc.load_gather(bounce_i, [jnp.minimum(iota + 1, 15)])
  dn = plsc.load_gather(bounce_i, [jnp.maximum(iota - 1, 0)])
  mask_last = active & ((iota == 15) | (up != idx_m))
  mask_first = active & ((iota == 0) | (dn != idx_m))
  t = plsc.cumsum(val_m)
  bounce_f[...] = t
  t_dn = plsc.load_gather(bounce_f, [jnp.maximum(iota - 1, 0)])
  t_dn = jnp.where(iota == 0, 0.0, t_dn)
  loc = idx_m - blk_base
  r = loc // _COLS
  c = loc - r * _COLS
  plsc.addupdate_scatter(blk_v, [r, c], t, mask=mask_last)
  plsc.addupdate_scatter(blk_v, [r, c], -t_dn, mask=mask_first)


def _sc_body(tens_hbm, idx_hbm, val_hbm, meta_hbm, out_hbm,
             blk0, blk1, pi0, pv0, pi1, pv1, ri, rv, meta_v,
             bounce_i, bounce_f,
             sem_in0, sem_in1, sem_out0, sem_out1,
             sem_pi0, sem_pi1, sem_pv0, sem_pv1):
  cid = lax.axis_index("c")
  sid = lax.axis_index("s")
  wid = sid * _NC + cid
  row_base = wid * _RROWS
  pltpu.sync_copy(meta_hbm.at[wid], meta_v)

  blks = (blk0, blk1)
  sem_ins = (sem_in0, sem_in1)
  sem_outs = (sem_out0, sem_out1)
  pbufs = ((pi0, pv0, sem_pi0, sem_pv0), (pi1, pv1, sem_pi1, sem_pv1))

  def get_se(b):
    mvec = meta_v[pl.ds(b * 16, 16)]
    return mvec[0], mvec[1]

  def pair_a(p):
    return jnp.minimum((p // 8) * 8, _K - _PCB)

  def in_copy(b, j):
    return pltpu.make_async_copy(
        tens_hbm.at[pl.ds(row_base + b * _BROWS, _BROWS), :],
        blks[j], sem_ins[j])

  def out_copy(b, j):
    return pltpu.make_async_copy(
        blks[j], out_hbm.at[pl.ds(row_base + b * _BROWS, _BROWS), :],
        sem_outs[j])

  def pair_copies(a, p):
    pi, pv, spi, spv = pbufs[p]
    return (pltpu.make_async_copy(idx_hbm.at[pl.ds(a, _PCB)], pi, spi),
            pltpu.make_async_copy(val_hbm.at[pl.ds(a, _PCB)], pv, spv))

  def consume(blk, idxb, valb, a, cstart, cend, blk_base):
    nvec = (cend - a + 15) // 16

    def vec_body(v, carry):
      o = v * 16
      idx16 = idxb[pl.ds(o, 16)]
      val16 = valb[pl.ds(o, 16)]
      pos = a + o + lax.iota(jnp.int32, 16)
      active = (pos >= cstart) & (pos < cend)
      _vec_update(blk, bounce_i, bounce_f, idx16, val16, active, blk_base)
      return carry

    lax.fori_loop(0, nvec, vec_body, 0)

  def half_blk(b, j, pa, p):
    sb, eb = get_se(b)

    @pl.when(b >= 1)
    def _():
      out_copy(b - 1, 1 - j).wait()

    @pl.when(b + 1 < _NBLK)
    def _():
      in_copy(b + 1, 1 - j).start()

    in_copy(b, j).wait()
    blk_base = (row_base + b * _BROWS) * _COLS
    covered = eb <= pa + _PCB

    @pl.when(covered)
    def _():
      pi, pv, _, _ = pbufs[p]
      consume(blks[j], pi, pv, pa, sb, eb, blk_base)

    @pl.when(jnp.logical_not(covered))
    def _():
      nchunks = (eb - sb + _PC - 1) // _PC

      def chunk_body(c, carry):
        cstart = sb + c * _PC
        cend = jnp.minimum(eb, cstart + _PC)
        ac = pair_a(cstart)
        pltpu.sync_copy(idx_hbm.at[pl.ds(ac, _PCB)], ri)
        pltpu.sync_copy(val_hbm.at[pl.ds(ac, _PCB)], rv)
        consume(blks[j], ri, rv, ac, cstart, cend, blk_base)
        return carry

      lax.fori_loop(0, jnp.maximum(nchunks, 0), chunk_body, 0)

    out_copy(b, j).start()

  in_copy(0, 0).start()
  sb0, _ = get_se(0)
  ci, cv = pair_copies(pair_a(sb0), 0)
  ci.start()
  cv.start()

  def group_step(g, carry):
    b0 = 2 * g
    p = lax.rem(g, 2)
    sbg, _ = get_se(b0)
    pa = pair_a(sbg)

    @pl.when(p == 0)
    def _():
      c0, c1 = pair_copies(0, 0)
      c0.wait()
      c1.wait()

    @pl.when(p == 1)
    def _():
      c0, c1 = pair_copies(0, 1)
      c0.wait()
      c1.wait()

    @pl.when(g + 1 < _NGRP)
    def _():
      sbn, _ = get_se(b0 + 2)

      @pl.when(p == 0)
      def _():
        c0, c1 = pair_copies(pair_a(sbn), 1)
        c0.start()
        c1.start()

      @pl.when(p == 1)
      def _():
        c0, c1 = pair_copies(pair_a(sbn), 0)
        c0.start()
        c1.start()

    @pl.when(p == 0)
    def _():
      half_blk(b0, 0, pa, 0)
      half_blk(b0 + 1, 1, pa, 0)

    @pl.when(p == 1)
    def _():
      half_blk(b0, 0, pa, 1)
      half_blk(b0 + 1, 1, pa, 1)

    return carry

  lax.fori_loop(0, _NGRP, group_step, 0)
  out_copy(_NBLK - 1, (_NBLK - 1) % 2).wait()


def kernel(tensor, values, indices):
  boundaries = (jnp.arange(_NW * _NBLK + 1, dtype=jnp.int32) * _BLK)
  bs = jnp.searchsorted(indices, boundaries, side="left").astype(jnp.int32)
  inter = jnp.stack([bs[:-1], bs[1:]], axis=1).reshape(_NW, _NBLK, 2)
  meta = (jnp.zeros((_NW, _NBLK, 16), dtype=jnp.int32)
          .at[:, :, :2].set(inter).reshape(_NW, _MROW))

  mesh = plsc.VectorSubcoreMesh(
      core_axis_name="c", subcore_axis_name="s",
      num_cores=_NC, num_subcores=_NS)
  run = pl.kernel(
      _sc_body,
      out_type=jax.ShapeDtypeStruct(_SHAPE, jnp.float32),
      mesh=mesh,
      compiler_params=pltpu.CompilerParams(needs_layout_passes=False),
      scratch_types=[
          pltpu.VMEM((_BROWS, _COLS), jnp.float32),
          pltpu.VMEM((_BROWS, _COLS), jnp.float32),
          pltpu.VMEM((_PCB,), jnp.int32),
          pltpu.VMEM((_PCB,), jnp.float32),
          pltpu.VMEM((_PCB,), jnp.int32),
          pltpu.VMEM((_PCB,), jnp.float32),
          pltpu.VMEM((_PCB,), jnp.int32),
          pltpu.VMEM((_PCB,), jnp.float32),
          pltpu.VMEM((_MROW,), jnp.int32),
          pltpu.VMEM((16,), jnp.int32),
          pltpu.VMEM((16,), jnp.float32),
          pltpu.SemaphoreType.DMA,
          pltpu.SemaphoreType.DMA,
          pltpu.SemaphoreType.DMA,
          pltpu.SemaphoreType.DMA,
          pltpu.SemaphoreType.DMA,
          pltpu.SemaphoreType.DMA,
          pltpu.SemaphoreType.DMA,
          pltpu.SemaphoreType.DMA,
      ],
  )
  return run(tensor, indices, values, meta)


# speedup vs baseline: 1.6531x; 1.0982x over previous
import jax
import jax.numpy as jnp
from jax import lax
from jax.experimental import pallas as pl
from jax.experimental.pallas import tpu as pltpu
from jax.experimental.pallas import tpu_sc as plsc

_SHAPE = (4096, 11008)
_COLS = _SHAPE[1]
_NUMEL = _SHAPE[0] * _SHAPE[1]
_NC, _NS = 2, 16
_NW = _NC * _NS
_RROWS = _SHAPE[0] // _NW
_BROWS = 4
_BLK = _BROWS * _COLS
_NBLK = _RROWS // _BROWS
_K = 1_000_000
_PC = 1536
_PCB = _PC + 16
_MROW = _NBLK * 16


def _vec_update(blk_v, bounce_i, bounce_f, idx16, val16, active, blk_base):
  iota = lax.iota(jnp.int32, 16)
  idx_m = jnp.where(active, idx16, -1)
  val_m = jnp.where(active, val16, 0.0)
  bounce_i[...] = idx_m
  up = plsc.load_gather(bounce_i, [jnp.minimum(iota + 1, 15)])
  dn = plsc.load_gather(bounce_i, [jnp.maximum(iota - 1, 0)])
  mask_last = active & ((iota == 15) | (up != idx_m))
  mask_first = active & ((iota == 0) | (dn != idx_m))
  t = plsc.cumsum(val_m)
  bounce_f[...] = t
  t_dn = plsc.load_gather(bounce_f, [jnp.maximum(iota - 1, 0)])
  t_dn = jnp.where(iota == 0, 0.0, t_dn)
  loc = idx_m - blk_base
  r = loc // _COLS
  c = loc - r * _COLS
  plsc.addupdate_scatter(blk_v, [r, c], t, mask=mask_last)
  plsc.addupdate_scatter(blk_v, [r, c], -t_dn, mask=mask_first)


class _BufSet:
  def __init__(self, blk, pi, pv, sem_in, sem_out, sem_pi, sem_pv):
    self.blk, self.pi, self.pv = blk, pi, pv
    self.sem_in, self.sem_out = sem_in, sem_out
    self.sem_pi, self.sem_pv = sem_pi, sem_pv


def _sc_body(tens_hbm, idx_hbm, val_hbm, meta_hbm, out_hbm,
             blk0, blk1, pi0, pv0, pi1, pv1, ri, rv, meta_v,
             bounce_i, bounce_f,
             sem_in0, sem_in1, sem_out0, sem_out1,
             sem_pi0, sem_pi1, sem_pv0, sem_pv1):
  cid = lax.axis_index("c")
  sid = lax.axis_index("s")
  wid = sid * _NC + cid
  row_base = wid * _RROWS
  pltpu.sync_copy(meta_hbm.at[wid], meta_v)

  set0 = _BufSet(blk0, pi0, pv0, sem_in0, sem_out0, sem_pi0, sem_pv0)
  set1 = _BufSet(blk1, pi1, pv1, sem_in1, sem_out1, sem_pi1, sem_pv1)

  def get_se(b):
    mvec = meta_v[pl.ds(b * 16, 16)]
    return mvec[0], mvec[1]

  def pair_a(p):
    return jnp.minimum((p // 8) * 8, _K - _PCB)

  def in_copy(b, s):
    return pltpu.make_async_copy(
        tens_hbm.at[pl.ds(row_base + b * _BROWS, _BROWS), :], s.blk, s.sem_in)

  def out_copy(b, s):
    return pltpu.make_async_copy(
        s.blk, out_hbm.at[pl.ds(row_base + b * _BROWS, _BROWS), :], s.sem_out)

  def pair_copies(a, s):
    return (pltpu.make_async_copy(idx_hbm.at[pl.ds(a, _PCB)], s.pi, s.sem_pi),
            pltpu.make_async_copy(val_hbm.at[pl.ds(a, _PCB)], s.pv, s.sem_pv))

  def issue_front(b, s):
    in_copy(b, s).start()
    sb, _ = get_se(b)
    ci, cv = pair_copies(pair_a(sb), s)
    ci.start()
    cv.start()

  def consume(blk, idxb, valb, a, cstart, cend, blk_base):
    nvec = (cend - a + 15) // 16

    def vec_body(v, carry):
      o = v * 16
      idx16 = idxb[pl.ds(o, 16)]
      val16 = valb[pl.ds(o, 16)]
      pos = a + o + lax.iota(jnp.int32, 16)
      active = (pos >= cstart) & (pos < cend)
      _vec_update(blk, bounce_i, bounce_f, idx16, val16, active, blk_base)
      return carry

    lax.fori_loop(0, nvec, vec_body, 0)

  def half(b, mine, other):
    sb, eb = get_se(b)

    @pl.when(b >= 1)
    def _():
      out_copy(b - 1, other).wait()

    @pl.when(b + 1 < _NBLK)
    def _():
      issue_front(b + 1, other)

    ci, cv = pair_copies(0, mine)
    ci.wait()
    cv.wait()
    in_copy(b, mine).wait()

    blk_base = (row_base + b * _BROWS) * _COLS
    a0 = pair_a(sb)
    consume(mine.blk, mine.pi, mine.pv, a0, sb, jnp.minimum(eb, sb + _PC),
            blk_base)
    nchunks = (eb - sb + _PC - 1) // _PC

    def chunk_body(c, carry):
      cstart = sb + c * _PC
      cend = jnp.minimum(eb, cstart + _PC)
      ac = pair_a(cstart)
      pltpu.sync_copy(idx_hbm.at[pl.ds(ac, _PCB)], ri)
      pltpu.sync_copy(val_hbm.at[pl.ds(ac, _PCB)], rv)
      consume(mine.blk, ri, rv, ac, cstart, cend, blk_base)
      return carry

    lax.fori_loop(1, jnp.maximum(nchunks, 1), chunk_body, 0)
    out_copy(b, mine).start()

  issue_front(0, set0)

  def pair_step(g, carry):
    b0 = 2 * g
    half(b0, set0, set1)

    @pl.when(b0 + 1 < _NBLK)
    def _():
      half(b0 + 1, set1, set0)

    return carry

  lax.fori_loop(0, (_NBLK + 1) // 2, pair_step, 0)
  out_copy(_NBLK - 1, set0 if (_NBLK - 1) % 2 == 0 else set1).wait()


def kernel(tensor, values, indices):
  boundaries = (jnp.arange(_NW * _NBLK + 1, dtype=jnp.int32) * _BLK)
  bs = jnp.searchsorted(indices, boundaries, side="left").astype(jnp.int32)
  inter = jnp.stack([bs[:-1], bs[1:]], axis=1).reshape(_NW, _NBLK, 2)
  meta = (jnp.zeros((_NW, _NBLK, 16), dtype=jnp.int32)
          .at[:, :, :2].set(inter).reshape(_NW, _MROW))

  mesh = plsc.VectorSubcoreMesh(
      core_axis_name="c", subcore_axis_name="s",
      num_cores=_NC, num_subcores=_NS)
  run = pl.kernel(
      _sc_body,
      out_type=jax.ShapeDtypeStruct(_SHAPE, jnp.float32),
      mesh=mesh,
      compiler_params=pltpu.CompilerParams(needs_layout_passes=False),
      scratch_types=[
          pltpu.VMEM((_BROWS, _COLS), jnp.float32),
          pltpu.VMEM((_BROWS, _COLS), jnp.float32),
          pltpu.VMEM((_PCB,), jnp.int32),
          pltpu.VMEM((_PCB,), jnp.float32),
          pltpu.VMEM((_PCB,), jnp.int32),
          pltpu.VMEM((_PCB,), jnp.float32),
          pltpu.VMEM((_PCB,), jnp.int32),
          pltpu.VMEM((_PCB,), jnp.float32),
          pltpu.VMEM((_MROW,), jnp.int32),
          pltpu.VMEM((16,), jnp.int32),
          pltpu.VMEM((16,), jnp.float32),
          pltpu.SemaphoreType.DMA,
          pltpu.SemaphoreType.DMA,
          pltpu.SemaphoreType.DMA,
          pltpu.SemaphoreType.DMA,
          pltpu.SemaphoreType.DMA,
          pltpu.SemaphoreType.DMA,
          pltpu.SemaphoreType.DMA,
          pltpu.SemaphoreType.DMA,
      ],
  )
  return run(tensor, indices, values, meta)
